# baseline (speedup 1.0000x reference)
import functools

import jax
import jax.numpy as jnp
from jax import lax
from jax.experimental import pallas as pl
from jax.experimental.pallas import tpu as pltpu
from jax.experimental.pallas import tpu_sc as plsc

_NC = 2
_NS = 16
_NW = _NC * _NS
_CH = 128
_LP = 128
_BC = 4096


def _dense_packed(xt, ict, w):
    d, n = xt.shape
    ng = (n + 2 * _BC - 1) // (2 * _BC)
    pr = ng * _BC

    def body(xt_ref, ict_ref, w_ref, o_ref):
        ct = xt_ref[...] * ict_ref[...]
        o_ref[:, :d] = lax.dot_general(
            ct[:, :_BC], w_ref[...], (((0,), (0,)), ((), ())),
            preferred_element_type=jnp.float32)
        o_ref[:, d:] = lax.dot_general(
            ct[:, _BC:], w_ref[...], (((0,), (0,)), ((), ())),
            preferred_element_type=jnp.float32)

    return pl.pallas_call(
        body,
        grid=(ng,),
        in_specs=[
            pl.BlockSpec((d, 2 * _BC), lambda g: (0, g)),
            pl.BlockSpec((d, 2 * _BC), lambda g: (0, g)),
            pl.BlockSpec((d, d), lambda g: (0, 0)),
        ],
        out_specs=pl.BlockSpec((_BC, _LP), lambda g: (g, 0)),
        out_shape=jax.ShapeDtypeStruct((pr, _LP), jnp.float32),
        compiler_params=pltpu.CompilerParams(
            dimension_semantics=("parallel",)),
    )(xt, ict, w)


@functools.lru_cache(maxsize=None)
def _sc_gather_fn(n_chunks_per_worker):
    nch = n_chunks_per_worker
    bpw = nch * _CH
    mesh = plsc.VectorSubcoreMesh(core_axis_name="c", subcore_axis_name="s")

    @functools.partial(
        pl.kernel,
        mesh=mesh,
        out_type=jax.ShapeDtypeStruct((_NW * bpw, _LP), jnp.float32),
        scratch_types=[
            pltpu.VMEM((nch, _CH), jnp.int32),
            pltpu.VMEM((bpw, _LP), jnp.float32),
            pltpu.SemaphoreType.DMA,
        ],
    )
    def sc_gather(table_hbm, idx_hbm, out_hbm, idx_v, rows_v, sem):
        wid = lax.axis_index("s") * _NC + lax.axis_index("c")
        pltpu.sync_copy(idx_hbm.at[pl.ds(wid * nch, nch)], idx_v)
        copies = []
        for j in range(nch):
            copies.append(pltpu.async_copy(
                table_hbm.at[idx_v.at[j]],
                rows_v.at[pl.ds(j * _CH, _CH)], sem))
        for c in copies:
            c.wait()
        pltpu.sync_copy(rows_v, out_hbm.at[pl.ds(wid * bpw, bpw)])

    return sc_gather


def kernel(x, input_constant, W, output_constant):
    n, d = x.shape
    b = output_constant.shape[0]
    assert b % (_NW * _CH) == 0
    nch = b // (_NW * _CH)

    xt = x.T
    ict = input_constant.T

    p = _dense_packed(xt, ict, W)

    idx = output_constant.astype(jnp.int32)
    grp = idx // (2 * _BC)
    r = idx % (2 * _BC)
    p_row = grp * _BC + (r % _BC)
    half = (r // _BC).astype(jnp.bool_)

    rows = _sc_gather_fn(nch)(p, p_row.reshape(b // _CH, _CH))
    return jnp.where(half[:, None], rows[:, d:], rows[:, :d])


# speedup vs baseline: 1.0344x; 1.0344x over previous
import functools

import jax
import jax.numpy as jnp
from jax import lax
from jax.experimental import pallas as pl
from jax.experimental.pallas import tpu as pltpu
from jax.experimental.pallas import tpu_sc as plsc

_NC = 2
_NS = 16
_NW = _NC * _NS
_CH = 128
_LP = 128
_BC = 4096


def _dense_packed(xt, ict, w):
    d, n = xt.shape
    ng = (n + 2 * _BC - 1) // (2 * _BC)
    pr = ng * _BC

    def body(xt_ref, ict_ref, w_ref, o_ref):
        ct = xt_ref[...] * ict_ref[...]
        ctp = jnp.concatenate([ct[:, :_BC], ct[:, _BC:]], axis=0)
        o_ref[...] = lax.dot_general(
            ctp, w_ref[...], (((0,), (0,)), ((), ())),
            preferred_element_type=jnp.float32)

    return pl.pallas_call(
        body,
        grid=(ng,),
        in_specs=[
            pl.BlockSpec((d, 2 * _BC), lambda g: (0, g)),
            pl.BlockSpec((d, 2 * _BC), lambda g: (0, g)),
            pl.BlockSpec((2 * d, _LP), lambda g: (0, 0)),
        ],
        out_specs=pl.BlockSpec((_BC, _LP), lambda g: (g, 0)),
        out_shape=jax.ShapeDtypeStruct((pr, _LP), jnp.float32),
        compiler_params=pltpu.CompilerParams(
            dimension_semantics=("parallel",)),
    )(xt, ict, w)


@functools.lru_cache(maxsize=None)
def _sc_gather_fn(n_chunks_per_worker):
    nch = n_chunks_per_worker
    bpw = nch * _CH
    mesh = plsc.VectorSubcoreMesh(core_axis_name="c", subcore_axis_name="s")

    @functools.partial(
        pl.kernel,
        mesh=mesh,
        out_type=jax.ShapeDtypeStruct((_NW * bpw, _LP), jnp.float32),
        scratch_types=[
            pltpu.VMEM((nch, _CH), jnp.int32),
            pltpu.VMEM((bpw, _LP), jnp.float32),
            pltpu.SemaphoreType.DMA,
        ],
    )
    def sc_gather(table_hbm, idx_hbm, out_hbm, idx_v, rows_v, sem):
        wid = lax.axis_index("s") * _NC + lax.axis_index("c")
        pltpu.sync_copy(idx_hbm.at[pl.ds(wid * nch, nch)], idx_v)
        copies = []
        for j in range(nch):
            copies.append(pltpu.async_copy(
                table_hbm.at[idx_v.at[j]],
                rows_v.at[pl.ds(j * _CH, _CH)], sem))
        for c in copies:
            c.wait()
        pltpu.sync_copy(rows_v, out_hbm.at[pl.ds(wid * bpw, bpw)])

    return sc_gather


def kernel(x, input_constant, W, output_constant):
    n, d = x.shape
    b = output_constant.shape[0]
    assert b % (_NW * _CH) == 0
    nch = b // (_NW * _CH)

    xt = x.T
    ict = input_constant.T

    w_big = jnp.zeros((2 * d, _LP), W.dtype)
    w_big = w_big.at[:d, :d].set(W).at[d:, d:].set(W)
    p = _dense_packed(xt, ict, w_big)

    idx = output_constant.astype(jnp.int32)
    grp = idx // (2 * _BC)
    r = idx % (2 * _BC)
    p_row = grp * _BC + (r % _BC)
    half = (r // _BC).astype(jnp.bool_)

    rows = _sc_gather_fn(nch)(p, p_row.reshape(b // _CH, _CH))
    return jnp.where(half[:, None], rows[:, d:], rows[:, :d])


# speedup vs baseline: 1.2176x; 1.1771x over previous
import functools

import jax
import jax.numpy as jnp
from jax import lax
from jax.experimental import pallas as pl
from jax.experimental.pallas import tpu as pltpu
from jax.experimental.pallas import tpu_sc as plsc

_NC = 2
_NS = 16
_NW = _NC * _NS
_CH = 128
_LP = 128
_BC = 16384


def _dense_rows(xt, ict, w):
    d, n = xt.shape
    grid = (n + _BC - 1) // _BC

    def body(xt_ref, ict_ref, w_ref, o_ref):
        ct = xt_ref[...] * ict_ref[...]
        o_ref[:, :d] = lax.dot_general(
            ct, w_ref[...], (((0,), (0,)), ((), ())),
            preferred_element_type=jnp.float32)

    return pl.pallas_call(
        body,
        grid=(grid,),
        in_specs=[
            pl.BlockSpec((d, _BC), lambda i: (0, i)),
            pl.BlockSpec((d, _BC), lambda i: (0, i)),
            pl.BlockSpec((d, d), lambda i: (0, 0)),
        ],
        out_specs=pl.BlockSpec((_BC, _LP), lambda i: (i, 0)),
        out_shape=jax.ShapeDtypeStruct((n, _LP), jnp.float32),
        compiler_params=pltpu.CompilerParams(
            dimension_semantics=("parallel",)),
    )(xt, ict, w)


@functools.lru_cache(maxsize=None)
def _sc_gather_fn(n_chunks_per_worker):
    nch = n_chunks_per_worker
    bpw = nch * _CH
    mesh = plsc.VectorSubcoreMesh(core_axis_name="c", subcore_axis_name="s")

    @functools.partial(
        pl.kernel,
        mesh=mesh,
        out_type=jax.ShapeDtypeStruct((_NW * bpw, _LP), jnp.float32),
        scratch_types=[
            pltpu.VMEM((nch, _CH), jnp.int32),
            pltpu.VMEM((bpw, _LP), jnp.float32),
            pltpu.SemaphoreType.DMA,
        ],
    )
    def sc_gather(table_hbm, idx_hbm, out_hbm, idx_v, rows_v, sem):
        wid = lax.axis_index("s") * _NC + lax.axis_index("c")
        pltpu.sync_copy(idx_hbm.at[pl.ds(wid * nch, nch)], idx_v)
        copies = []
        for j in range(nch):
            copies.append(pltpu.async_copy(
                table_hbm.at[idx_v.at[j]],
                rows_v.at[pl.ds(j * _CH, _CH)], sem))
        for c in copies:
            c.wait()
        pltpu.sync_copy(rows_v, out_hbm.at[pl.ds(wid * bpw, bpw)])

    return sc_gather


def kernel(x, input_constant, W, output_constant):
    n, d = x.shape
    b = output_constant.shape[0]
    assert b % (_NW * _CH) == 0
    nch = b // (_NW * _CH)

    xt = x.T
    ict = input_constant.T

    r = _dense_rows(xt, ict, W)
    idx = output_constant.astype(jnp.int32)
    rows = _sc_gather_fn(nch)(r, idx.reshape(b // _CH, _CH))
    return rows[:, :d]
